# baseline (speedup 1.0000x reference)
import jax
import jax.numpy as jnp
from jax import lax
from jax.experimental import pallas as pl
from jax.experimental.pallas import tpu as pltpu
from jax.experimental.pallas import tpu_sc as plsc

NC = 2
NS = 16
NW = NC * NS

L = 16
NB = 32
XW = 128
_SLICES = (0, 16, 32, 48, 64, 80, 84)


def _sc_body(x_hbm, tab_hbm, out_hbm, xv, idx_v, rows_v, off_v, rid_v,
             tab_sh, sem):
    wid = lax.axis_index("s") * NC + lax.axis_index("c")
    n_total = x_hbm.shape[0]
    n_genes = out_hbm.shape[0] // n_total
    nb_w = n_total // NW
    n_chunks = nb_w // NB

    @pl.when(lax.axis_index("s") == 0)
    def _stage():
        pltpu.sync_copy(tab_hbm, tab_sh)

    iota = lax.iota(jnp.int32, L)
    for st in _SLICES:
        off_v[pl.ds(st, L)] = (iota + st) * 3

    plsc.subcore_barrier()

    def chunk(i, carry):
        nb0 = wid * nb_w + i * NB
        rid_v[pl.ds(0, L)] = nb0 + iota
        rid_v[pl.ds(L, L)] = nb0 + L + iota
        pltpu.async_copy(x_hbm.at[rid_v], xv, sem).wait()
        for r in range(NB):
            for st in _SLICES:
                sl = pl.ds(st, L)
                idx_v[r, sl] = xv[r, sl] + off_v[sl]
        cps = [
            pltpu.async_copy(
                tab_sh.at[idx_v.at[r]],
                rows_v.at[pl.ds(r * n_genes, n_genes)],
                sem,
            )
            for r in range(NB)
        ]
        for c in cps:
            c.wait()
        pltpu.sync_copy(rows_v, out_hbm.at[pl.ds(nb0 * n_genes, NB * n_genes)])
        return carry

    lax.fori_loop(0, n_chunks, chunk, 0)


def _pad_body(x_ref, o_ref):
    blk = x_ref[...]
    z = jnp.zeros((blk.shape[0], XW - blk.shape[1]), blk.dtype)
    o_ref[...] = jnp.concatenate([blk, z], axis=1)


def kernel(x, emb_tables):
    n, g = x.shape
    _, cat, h = emb_tables.shape
    rows = n * g
    blk = 2048
    xp = pl.pallas_call(
        _pad_body,
        grid=(n // blk,),
        in_specs=[pl.BlockSpec((blk, g), lambda i: (i, 0))],
        out_specs=pl.BlockSpec((blk, XW), lambda i: (i, 0)),
        out_shape=jax.ShapeDtypeStruct((n, XW), jnp.int32),
    )(x)
    tab = emb_tables.reshape(g * cat, h)

    mesh = plsc.VectorSubcoreMesh(core_axis_name="c", subcore_axis_name="s")
    out = pl.kernel(
        _sc_body,
        out_type=jax.ShapeDtypeStruct((rows, h), jnp.float32),
        mesh=mesh,
        scratch_types=[
            pltpu.VMEM((NB, XW), jnp.int32),
            pltpu.VMEM((NB, g), jnp.int32),
            pltpu.VMEM((NB * g, h), jnp.float32),
            pltpu.VMEM((g,), jnp.int32),
            pltpu.VMEM((NB,), jnp.int32),
            pltpu.VMEM_SHARED((g * cat, h), jnp.float32),
            pltpu.SemaphoreType.DMA,
        ],
        compiler_params=pltpu.CompilerParams(use_tc_tiling_on_sc=False),
    )(xp, tab)
    return out.reshape(n, g, h)


# speedup vs baseline: 3.1359x; 3.1359x over previous
import jax
import jax.numpy as jnp
from jax import lax
from jax.experimental import pallas as pl
from jax.experimental.pallas import tpu as pltpu
from jax.experimental.pallas import tpu_sc as plsc

NC = 2
NS = 16
NW = NC * NS

L = 16
NB = 32
XW = 128
_SLICES = (0, 16, 32, 48, 64, 80, 84)


def _sc_body(x_hbm, tab_hbm, out_hbm, xv, idx_v, rows_v, off_v, rid_v,
             tab_sh, sem):
    wid = lax.axis_index("s") * NC + lax.axis_index("c")
    n_total = x_hbm.shape[0]
    n_genes = out_hbm.shape[1]
    nb_w = n_total // NW
    n_chunks = nb_w // NB

    @pl.when(lax.axis_index("s") == 0)
    def _stage():
        pltpu.sync_copy(tab_hbm, tab_sh)

    iota = lax.iota(jnp.int32, L)
    for st in _SLICES:
        off_v[pl.ds(st, L)] = (iota + st) * 3

    plsc.subcore_barrier()

    def chunk(i, carry):
        nb0 = wid * nb_w + i * NB
        rid_v[pl.ds(0, L)] = nb0 + iota
        rid_v[pl.ds(L, L)] = nb0 + L + iota
        pltpu.async_copy(x_hbm.at[rid_v], xv, sem).wait()
        for r in range(NB):
            for st in _SLICES:
                sl = pl.ds(st, L)
                idx_v[r, sl] = xv[r, sl] + off_v[sl]
        cps = [
            pltpu.async_copy(tab_sh.at[idx_v.at[r]], rows_v.at[r], sem)
            for r in range(NB)
        ]
        for c in cps:
            c.wait()
        pltpu.sync_copy(rows_v, out_hbm.at[pl.ds(nb0, NB)])
        return carry

    lax.fori_loop(0, n_chunks, chunk, 0)


def _pad_body(x_ref, o_ref):
    blk = x_ref[...]
    z = jnp.zeros((blk.shape[0], XW - blk.shape[1]), blk.dtype)
    o_ref[...] = jnp.concatenate([blk, z], axis=1)


def kernel(x, emb_tables):
    n, g = x.shape
    _, cat, h = emb_tables.shape
    rows = n * g
    blk = 2048
    xp = pl.pallas_call(
        _pad_body,
        grid=(n // blk,),
        in_specs=[pl.BlockSpec((blk, g), lambda i: (i, 0))],
        out_specs=pl.BlockSpec((blk, XW), lambda i: (i, 0)),
        out_shape=jax.ShapeDtypeStruct((n, XW), jnp.int32),
    )(x)
    tab = emb_tables.reshape(g * cat, h)

    mesh = plsc.VectorSubcoreMesh(core_axis_name="c", subcore_axis_name="s")
    out = pl.kernel(
        _sc_body,
        out_type=jax.ShapeDtypeStruct((n, g, h), jnp.float32),
        mesh=mesh,
        scratch_types=[
            pltpu.VMEM((NB, XW), jnp.int32),
            pltpu.VMEM((NB, g), jnp.int32),
            pltpu.VMEM((NB, g, h), jnp.float32),
            pltpu.VMEM((g,), jnp.int32),
            pltpu.VMEM((NB,), jnp.int32),
            pltpu.VMEM_SHARED((g * cat, h), jnp.float32),
            pltpu.SemaphoreType.DMA,
        ],
        compiler_params=pltpu.CompilerParams(use_tc_tiling_on_sc=False),
    )(xp, tab)
    return out
